# baseline (speedup 1.0000x reference)
import functools

import jax
import jax.numpy as jnp
from jax import lax
from jax.experimental import pallas as pl
from jax.experimental.pallas import tpu as pltpu
from jax.experimental.pallas import tpu_sc as plsc

_VOCAB = 50257
_N_POS = 1024
_D = 768
_B = 16
_S = 1024
_N = _B * _S
_NC = 2
_NS = 16
_NW = _NC * _NS
_CW = _S // _NW
_H = _CW // 2
_LANES = _D // 16
_NBUF = 4


def _make_emb_kernel():
  mesh = plsc.VectorSubcoreMesh(core_axis_name="c", subcore_axis_name="s")

  @functools.partial(
      pl.kernel,
      mesh=mesh,
      out_type=jax.ShapeDtypeStruct((_N, _D), jnp.float32),
      scratch_types=(
          [pltpu.VMEM((_B, _CW), jnp.int32),
           pltpu.VMEM((_CW, _D), jnp.float32)]
          + [pltpu.VMEM((_CW, _D), jnp.float32)] * _NBUF
          + [pltpu.SemaphoreType.DMA] * (2 + 3 * _NBUF)
      ),
  )
  def emb(idx_hbm, tok_hbm, pos_hbm, out_hbm, idx_v, pos_v, *rest):
    toks = list(rest[:_NBUF])
    semi, semp = rest[_NBUF], rest[_NBUF + 1]
    sgs = list(rest[_NBUF + 2:_NBUF + 2 + _NBUF])
    sgs2 = list(rest[_NBUF + 2 + _NBUF:_NBUF + 2 + 2 * _NBUF])
    sos = list(rest[_NBUF + 2 + 2 * _NBUF:])
    wid = lax.axis_index("s") * _NC + lax.axis_index("c")
    col0 = wid * _CW
    his = [
        pltpu.async_copy(
            idx_hbm.at[pl.ds(b * _S + col0, _CW)], idx_v.at[b], semi)
        for b in range(_B)
    ]
    hp = pltpu.async_copy(pos_hbm.at[pl.ds(col0, _CW)], pos_v, semp)
    g = [None] * _NBUF
    o = [None] * _NBUF
    def gather(c, k):
      h1 = pltpu.async_copy(
          tok_hbm.at[idx_v.at[c, pl.ds(0, _H)]],
          toks[k].at[pl.ds(0, _H)], sgs[k])
      h2 = pltpu.async_copy(
          tok_hbm.at[idx_v.at[c, pl.ds(_H, _H)]],
          toks[k].at[pl.ds(_H, _H)], sgs2[k])
      return (h1, h2)

    for h in his:
      h.wait()
    for c in range(2):
      g[c] = gather(c, c)
    hp.wait()
    for b in range(_B):
      cur = b % _NBUF
      g[cur][0].wait()
      c = b + 2
      if c < _B:
        k = c % _NBUF
        if o[k] is not None:
          o[k].wait()
        g[k] = gather(c, k)
      tok = toks[cur]

      def add_row(r, _, tok=tok):
        for j in range(_LANES):
          sl = pl.ds(j * 16, 16)
          tok[r, sl] = tok[r, sl] + pos_v[r, sl]
        return ()

      lax.fori_loop(0, _H, add_row, ())
      g[cur][1].wait()
      lax.fori_loop(_H, _CW, add_row, ())
      o[cur] = pltpu.async_copy(
          tok, out_hbm.at[pl.ds(b * _S + col0, _CW)], sos[cur])
    for k in range(_NBUF):
      if o[k] is not None:
        o[k].wait()

  return emb


_emb = _make_emb_kernel()


def kernel(input, token_table, pos_table):
  idx = input.reshape(_N).astype(jnp.int32)
  out = _emb(idx, token_table, pos_table)
  return out.reshape(_B, _S, _D)


# speedup vs baseline: 1.0554x; 1.0554x over previous
import functools

import jax
import jax.numpy as jnp
from jax import lax
from jax.experimental import pallas as pl
from jax.experimental.pallas import tpu as pltpu
from jax.experimental.pallas import tpu_sc as plsc

_VOCAB = 50257
_N_POS = 1024
_D = 768
_B = 16
_S = 1024
_N = _B * _S
_NC = 2
_NS = 16
_NW = _NC * _NS
_CW = _S // _NW
_H = _CW // 2
_LANES = _D // 16
_NBUF = 4


def _make_emb_kernel():
  mesh = plsc.VectorSubcoreMesh(core_axis_name="c", subcore_axis_name="s")

  @functools.partial(
      pl.kernel,
      mesh=mesh,
      out_type=jax.ShapeDtypeStruct((_N, _D), jnp.float32),
      scratch_types=(
          [pltpu.VMEM((_B, _CW), jnp.int32),
           pltpu.VMEM((_CW, _D), jnp.float32)]
          + [pltpu.VMEM((_CW, _D), jnp.float32)] * _NBUF
          + [pltpu.SemaphoreType.DMA] * (2 + 2 * _NBUF)
      ),
  )
  def emb(idx_hbm, tok_hbm, pos_hbm, out_hbm, idx_v, pos_v, *rest):
    toks = list(rest[:_NBUF])
    semi, semp = rest[_NBUF], rest[_NBUF + 1]
    sgs = list(rest[_NBUF + 2:_NBUF + 2 + _NBUF])
    sos = list(rest[_NBUF + 2 + _NBUF:])
    wid = lax.axis_index("s") * _NC + lax.axis_index("c")
    col0 = wid * _CW
    his = [
        pltpu.async_copy(
            idx_hbm.at[pl.ds(b * _S + col0, _CW)], idx_v.at[b], semi)
        for b in range(_B)
    ]
    hp = pltpu.async_copy(pos_hbm.at[pl.ds(col0, _CW)], pos_v, semp)
    g = [None] * _NBUF
    o = [None] * _NBUF
    for h in his:
      h.wait()
    for c in range(2):
      g[c] = pltpu.async_copy(tok_hbm.at[idx_v.at[c]], toks[c], sgs[c])
    hp.wait()
    for b in range(_B):
      cur = b % _NBUF
      g[cur].wait()
      c = b + 2
      if c < _B:
        k = c % _NBUF
        if o[k] is not None:
          o[k].wait()
        g[k] = pltpu.async_copy(tok_hbm.at[idx_v.at[c]], toks[k], sgs[k])
      tok = toks[cur]

      def add_row(r, _, tok=tok):
        for j in range(_LANES):
          sl = pl.ds(j * 16, 16)
          tok[r, sl] = tok[r, sl] + pos_v[r, sl]
        return ()

      lax.fori_loop(0, _CW, add_row, ())
      o[cur] = pltpu.async_copy(
          tok, out_hbm.at[pl.ds(b * _S + col0, _CW)], sos[cur])
    for k in range(_NBUF):
      if o[k] is not None:
        o[k].wait()

  return emb


_emb = _make_emb_kernel()


def kernel(input, token_table, pos_table):
  idx = input.reshape(_N).astype(jnp.int32)
  out = _emb(idx, token_table, pos_table)
  return out.reshape(_B, _S, _D)
